# Initial kernel scaffold; baseline (speedup 1.0000x reference)
import jax
import jax.numpy as jnp
from jax.experimental import pallas as pl


def kernel(ind_2, px, i1, diff, W_pi_i, W_pi_j, W_dot_i, W_dot_j):
    raise NotImplementedError("write your pallas kernel here")


# speedup vs baseline: 7.4085x; 7.4085x over previous
import jax
import jax.numpy as jnp
from jax import lax
from jax.experimental import pallas as pl
from jax.experimental.pallas import tpu as pltpu
from jax.experimental.pallas import tpu_sc as plsc

N = 10000
E = 160000
C = 128
CHQ = 32
NC = 2
NS = 16
K = 40
EPT = E // NS
NCHK = EPT // K
RPT = N // NS
ZR = 25

f32 = jnp.float32
i32 = jnp.int32


def _tabs_body(px_ref, wi_ref, wj_ref, *outs):
    x = px_ref[...]
    yi = jnp.dot(x, wi_ref[...], preferred_element_type=f32)
    yj = jnp.dot(x, wj_ref[...], preferred_element_type=f32)
    for q in range(4):
        outs[q][...] = yi[:, q * CHQ:(q + 1) * CHQ]
        outs[4 + q][...] = yj[:, q * CHQ:(q + 1) * CHQ]


def _make_tabs(pxf, wi, wj):
    R = 600
    grid = (pxf.shape[0] // R,)
    outs = [jax.ShapeDtypeStruct((pxf.shape[0], CHQ), f32)] * 8
    return pl.pallas_call(
        _tabs_body,
        grid=grid,
        in_specs=[
            pl.BlockSpec((R, C), lambda i: (i, 0)),
            pl.BlockSpec((C, C), lambda i: (0, 0)),
            pl.BlockSpec((C, C), lambda i: (0, 0)),
        ],
        out_specs=[pl.BlockSpec((R, CHQ), lambda i: (i, 0))] * 8,
        out_shape=outs,
    )(pxf, wi, wj)


def _dot_body(pxn_ref, wi_ref, wj_ref, out_ref):
    Rn = pxn_ref.shape[0]
    x = pxn_ref[...].reshape(Rn * 3, C)
    a = jnp.dot(x, wi_ref[...], preferred_element_type=f32)
    b = jnp.dot(x, wj_ref[...], preferred_element_type=f32)
    out_ref[...] = (a * b).reshape(Rn, 3, C).sum(axis=1)


def _make_dot(pxn, wi, wj):
    Rn = 200
    grid = (N // Rn,)
    return pl.pallas_call(
        _dot_body,
        grid=grid,
        in_specs=[
            pl.BlockSpec((Rn, 3, C), lambda i: (i, 0, 0)),
            pl.BlockSpec((C, C), lambda i: (0, 0)),
            pl.BlockSpec((C, C), lambda i: (0, 0)),
        ],
        out_specs=pl.BlockSpec((Rn, C), lambda i: (i, 0)),
        out_shape=jax.ShapeDtypeStruct((N, C), f32),
    )(pxn, wi, wj)


def _sc_body(indi_hbm, indj_hbm, i1_hbm, diff_hbm,
             ti0, ti1, ti2, ti3, tj0, tj1, tj2, tj3,
             ix_hbm, pxnew_hbm,
             idxi_v, idxj_v, gi_v, gj_v, i1_v, diff_v, ixb_v, zb_v,
             acc_sh, sem_gi, sem_gj, sem_i1, sem_df):
    cid = lax.axis_index("c")
    sid = lax.axis_index("s")
    r0 = sid * RPT

    def zrow(r, _):
        for x in range(3):
            for qq in range(CHQ // 16):
                zb_v[r, x, pl.ds(qq * 16, 16)] = jnp.zeros((16,), f32)
        return 0
    lax.fori_loop(0, ZR, zrow, 0)

    pltpu.sync_copy(indi_hbm.at[sid], idxi_v)
    pltpu.sync_copy(indj_hbm.at[sid], idxj_v)

    def edge_body(e, _):
        for x in range(3):
            dv = plsc.load_gather(
                diff_v,
                [jnp.full((16,), e, i32), jnp.full((16,), x, i32)])
            for qq in range(CHQ // 16):
                sl = pl.ds(qq * 16, 16)
                g = gi_v[e, x, sl] + gj_v[e, x, sl] + dv
                ixb_v[e, x, sl] = g * i1_v[e, sl]
        return 0

    def do_chunk(t, tab_i, tab_j, coff):
        ii = idxi_v.at[t]
        ij = idxj_v.at[t]
        ci1 = pltpu.async_copy(
            i1_hbm.at[sid, t, :, pl.ds(coff, CHQ)], i1_v, sem_i1)
        cdf = pltpu.async_copy(diff_hbm.at[sid, t], diff_v, sem_df)
        cgi = pltpu.async_copy(tab_i.at[ii], gi_v, sem_gi)
        cgj = pltpu.async_copy(tab_j.at[ij], gj_v, sem_gj)
        cgi.wait()
        cgj.wait()
        ci1.wait()
        cdf.wait()
        lax.fori_loop(0, K, edge_body, 0)
        pltpu.sync_copy(ixb_v, ix_hbm.at[sid, t, :, :, pl.ds(coff, CHQ)])
        pltpu.sync_copy(ixb_v, acc_sh.at[ii], add=True)

    for h in range(2):
        for z in range(RPT // ZR):
            pltpu.sync_copy(zb_v, acc_sh.at[pl.ds(r0 + z * ZR, ZR)])
        plsc.subcore_barrier()

        tabs = ((ti0, tj0, 0), (ti1, tj1, CHQ)) if h == 0 else \
               ((ti2, tj2, 2 * CHQ), (ti3, tj3, 3 * CHQ))

        def chunk_body(t, _):
            @pl.when(cid == 0)
            def _():
                do_chunk(t, tabs[0][0], tabs[0][1], tabs[0][2])

            @pl.when(cid == 1)
            def _():
                do_chunk(t, tabs[1][0], tabs[1][1], tabs[1][2])
            return 0

        lax.fori_loop(0, NCHK, chunk_body, 0)
        plsc.subcore_barrier()

        @pl.when(cid == 0)
        def _():
            pltpu.sync_copy(
                acc_sh.at[pl.ds(r0, RPT)],
                pxnew_hbm.at[pl.ds(r0, RPT), :, pl.ds(tabs[0][2], CHQ)])

        @pl.when(cid == 1)
        def _():
            pltpu.sync_copy(
                acc_sh.at[pl.ds(r0, RPT)],
                pxnew_hbm.at[pl.ds(r0, RPT), :, pl.ds(tabs[1][2], CHQ)])


def _run_sc(indi, indj, i1r, diffp, tabs):
    mesh = plsc.VectorSubcoreMesh(core_axis_name="c", subcore_axis_name="s")
    k = pl.kernel(
        _sc_body,
        out_type=[
            jax.ShapeDtypeStruct((NS, NCHK, K, 3, C), f32),
            jax.ShapeDtypeStruct((N, 3, C), f32),
        ],
        mesh=mesh,
        compiler_params=pltpu.CompilerParams(use_tc_tiling_on_sc=False,
                                             needs_layout_passes=False),
        scratch_types=[
            pltpu.VMEM((NCHK, K), i32),
            pltpu.VMEM((NCHK, K), i32),
            pltpu.VMEM((K, 3, CHQ), f32),
            pltpu.VMEM((K, 3, CHQ), f32),
            pltpu.VMEM((K, CHQ), f32),
            pltpu.VMEM((K, 8), f32),
            pltpu.VMEM((K, 3, CHQ), f32),
            pltpu.VMEM((ZR, 3, CHQ), f32),
            pltpu.VMEM_SHARED((N, 3, CHQ), f32),
            pltpu.SemaphoreType.DMA,
            pltpu.SemaphoreType.DMA,
            pltpu.SemaphoreType.DMA,
            pltpu.SemaphoreType.DMA,
        ],
    )
    return k(indi, indj, i1r, diffp, *tabs)


def kernel(ind_2, px, i1, diff, W_pi_i, W_pi_j, W_dot_i, W_dot_j):
    ind_i = ind_2[:, 0].reshape(NS, NCHK, K)
    ind_j = ind_2[:, 1].reshape(NS, NCHK, K)
    i1r = i1.reshape(NS, NCHK, K, C)
    diffp = jnp.pad(diff, ((0, 0), (0, 5))).reshape(NS, NCHK, K, 8)

    pxf = px.reshape(N * 3, C)
    tabs = _make_tabs(pxf, W_pi_i, W_pi_j)
    tabs = [t.reshape(N, 3, CHQ) for t in tabs]

    ix, px_new = _run_sc(ind_i, ind_j, i1r, diffp, tabs)
    ix = ix.reshape(E, 3, C)

    dotted = _make_dot(px_new, W_dot_i, W_dot_j)
    return px_new, ix, dotted


# speedup vs baseline: 9.0576x; 1.2226x over previous
import jax
import jax.numpy as jnp
from jax import lax
from jax.experimental import pallas as pl
from jax.experimental.pallas import tpu as pltpu
from jax.experimental.pallas import tpu_sc as plsc

N = 10000
E = 160000
C = 128
CHQ = 32
NC = 2
NS = 16
K = 40
EPT = E // NS
NCHK = EPT // K
RPT = N // NS
ZR = 25

f32 = jnp.float32
i32 = jnp.int32


def _tabs_body(px_ref, wi_ref, wj_ref, *outs):
    x = px_ref[...]
    yi = jnp.dot(x, wi_ref[...], preferred_element_type=f32)
    yj = jnp.dot(x, wj_ref[...], preferred_element_type=f32)
    for q in range(4):
        outs[q][...] = yi[:, q * CHQ:(q + 1) * CHQ]
        outs[4 + q][...] = yj[:, q * CHQ:(q + 1) * CHQ]


def _make_tabs(pxf, wi, wj):
    R = 600
    grid = (pxf.shape[0] // R,)
    outs = [jax.ShapeDtypeStruct((pxf.shape[0], CHQ), f32)] * 8
    return pl.pallas_call(
        _tabs_body,
        grid=grid,
        in_specs=[
            pl.BlockSpec((R, C), lambda i: (i, 0)),
            pl.BlockSpec((C, C), lambda i: (0, 0)),
            pl.BlockSpec((C, C), lambda i: (0, 0)),
        ],
        out_specs=[pl.BlockSpec((R, CHQ), lambda i: (i, 0))] * 8,
        out_shape=outs,
    )(pxf, wi, wj)


def _dot_body(pxn_ref, wi_ref, wj_ref, out_ref):
    Rn = pxn_ref.shape[0]
    x = pxn_ref[...].reshape(Rn * 3, C)
    a = jnp.dot(x, wi_ref[...], preferred_element_type=f32)
    b = jnp.dot(x, wj_ref[...], preferred_element_type=f32)
    out_ref[...] = (a * b).reshape(Rn, 3, C).sum(axis=1)


def _make_dot(pxn, wi, wj):
    Rn = 200
    grid = (N // Rn,)
    return pl.pallas_call(
        _dot_body,
        grid=grid,
        in_specs=[
            pl.BlockSpec((Rn, 3, C), lambda i: (i, 0, 0)),
            pl.BlockSpec((C, C), lambda i: (0, 0)),
            pl.BlockSpec((C, C), lambda i: (0, 0)),
        ],
        out_specs=pl.BlockSpec((Rn, C), lambda i: (i, 0)),
        out_shape=jax.ShapeDtypeStruct((N, C), f32),
    )(pxn, wi, wj)


def _sc_body(indi_hbm, indj_hbm, i1_hbm, diff_hbm,
             ti0, ti1, ti2, ti3, tj0, tj1, tj2, tj3,
             ix_hbm, pxnew_hbm,
             idxi_v, idxj_v, gi_v, gj_v, i1_v, diff_v, ixb_v, zb_v,
             acc_sh, sem_in0, sem_in1):
    cid = lax.axis_index("c")
    sid = lax.axis_index("s")
    r0 = sid * RPT
    sem_in = (sem_in0, sem_in1)

    def zrow(r, _):
        for x in range(3):
            for qq in range(CHQ // 16):
                zb_v[r, x, pl.ds(qq * 16, 16)] = jnp.zeros((16,), f32)
        return 0
    lax.fori_loop(0, ZR, zrow, 0)

    pltpu.sync_copy(indi_hbm.at[sid], idxi_v)
    pltpu.sync_copy(indj_hbm.at[sid], idxj_v)

    def make_edge_body(s):
        def edge_body(e, _):
            for x in range(3):
                dv = plsc.load_gather(
                    diff_v,
                    [jnp.full((16,), s, i32), jnp.full((16,), e, i32),
                     jnp.full((16,), x, i32)])
                for qq in range(CHQ // 16):
                    sl = pl.ds(qq * 16, 16)
                    g = gi_v[s, e, x, sl] + gj_v[s, e, x, sl] + dv
                    ixb_v[s, e, x, sl] = g * i1_v[s, e, sl]
            return 0
        return edge_body

    edge_bodies = (make_edge_body(0), make_edge_body(1))

    def in_copies(t, s, tab_i, tab_j, coff):
        return (
            (tab_i.at[idxi_v.at[t]], gi_v.at[s], sem_in[s]),
            (tab_j.at[idxj_v.at[t]], gj_v.at[s], sem_in[s]),
            (i1_hbm.at[sid, t, :, pl.ds(coff, CHQ)], i1_v.at[s], sem_in[s]),
            (diff_hbm.at[sid, t], diff_v.at[s], sem_in[s]),
        )

    def issue(cs):
        for src, dst, sem in cs:
            pltpu.async_copy(src, dst, sem)

    def drain(cs):
        for src, dst, sem in cs:
            pltpu.make_async_copy(src, dst, sem).wait()

    def run_pass(tab_i, tab_j, coff):
        issue(in_copies(0, 0, tab_i, tab_j, coff))

        def body2(tt, _):
            t0 = 2 * tt
            for s in range(2):
                t = t0 + s
                drain(in_copies(t, s, tab_i, tab_j, coff))
                nxt = t + 1

                @pl.when(nxt < NCHK)
                def _():
                    issue(in_copies(nxt, 1 - s, tab_i, tab_j, coff))

                lax.fori_loop(0, K, edge_bodies[s], 0)
                pltpu.sync_copy(ixb_v.at[s],
                                ix_hbm.at[sid, t, :, :, pl.ds(coff, CHQ)])
                pltpu.sync_copy(ixb_v.at[s], acc_sh.at[idxi_v.at[t]],
                                add=True)
            return 0

        lax.fori_loop(0, NCHK // 2, body2, 0)

    for h in range(2):
        for z in range(RPT // ZR):
            pltpu.sync_copy(zb_v, acc_sh.at[pl.ds(r0 + z * ZR, ZR)])
        plsc.subcore_barrier()

        tabs = ((ti0, tj0, 0), (ti1, tj1, CHQ)) if h == 0 else \
               ((ti2, tj2, 2 * CHQ), (ti3, tj3, 3 * CHQ))

        @pl.when(cid == 0)
        def _():
            run_pass(tabs[0][0], tabs[0][1], tabs[0][2])

        @pl.when(cid == 1)
        def _():
            run_pass(tabs[1][0], tabs[1][1], tabs[1][2])

        plsc.subcore_barrier()

        @pl.when(cid == 0)
        def _():
            pltpu.sync_copy(
                acc_sh.at[pl.ds(r0, RPT)],
                pxnew_hbm.at[pl.ds(r0, RPT), :, pl.ds(tabs[0][2], CHQ)])

        @pl.when(cid == 1)
        def _():
            pltpu.sync_copy(
                acc_sh.at[pl.ds(r0, RPT)],
                pxnew_hbm.at[pl.ds(r0, RPT), :, pl.ds(tabs[1][2], CHQ)])


def _run_sc(indi, indj, i1r, diffp, tabs):
    mesh = plsc.VectorSubcoreMesh(core_axis_name="c", subcore_axis_name="s")
    k = pl.kernel(
        _sc_body,
        out_type=[
            jax.ShapeDtypeStruct((NS, NCHK, K, 3, C), f32),
            jax.ShapeDtypeStruct((N, 3, C), f32),
        ],
        mesh=mesh,
        compiler_params=pltpu.CompilerParams(use_tc_tiling_on_sc=False,
                                             needs_layout_passes=False),
        scratch_types=[
            pltpu.VMEM((NCHK, K), i32),
            pltpu.VMEM((NCHK, K), i32),
            pltpu.VMEM((2, K, 3, CHQ), f32),
            pltpu.VMEM((2, K, 3, CHQ), f32),
            pltpu.VMEM((2, K, CHQ), f32),
            pltpu.VMEM((2, K, 8), f32),
            pltpu.VMEM((2, K, 3, CHQ), f32),
            pltpu.VMEM((ZR, 3, CHQ), f32),
            pltpu.VMEM_SHARED((N, 3, CHQ), f32),
            pltpu.SemaphoreType.DMA,
            pltpu.SemaphoreType.DMA,
        ],
    )
    return k(indi, indj, i1r, diffp, *tabs)


def kernel(ind_2, px, i1, diff, W_pi_i, W_pi_j, W_dot_i, W_dot_j):
    ind_i = ind_2[:, 0].reshape(NS, NCHK, K)
    ind_j = ind_2[:, 1].reshape(NS, NCHK, K)
    i1r = i1.reshape(NS, NCHK, K, C)
    diffp = jnp.pad(diff, ((0, 0), (0, 5))).reshape(NS, NCHK, K, 8)

    pxf = px.reshape(N * 3, C)
    tabs = _make_tabs(pxf, W_pi_i, W_pi_j)
    tabs = [t.reshape(N, 3, CHQ) for t in tabs]

    ix, px_new = _run_sc(ind_i, ind_j, i1r, diffp, tabs)
    ix = ix.reshape(E, 3, C)

    dotted = _make_dot(px_new, W_dot_i, W_dot_j)
    return px_new, ix, dotted


# speedup vs baseline: 11.1757x; 1.2338x over previous
import jax
import jax.numpy as jnp
from jax import lax
from jax.experimental import pallas as pl
from jax.experimental.pallas import tpu as pltpu
from jax.experimental.pallas import tpu_sc as plsc

N = 10000
E = 160000
C = 128
CHQ = 32
NC = 2
NS = 16
K = 40
EPT = E // NS
NCHK = EPT // K
RPT = N // NS
ZR = 25

f32 = jnp.float32
i32 = jnp.int32


def _tabs_body(px_ref, wi_ref, wj_ref, *outs):
    x = px_ref[...]
    R = x.shape[0]
    yi = jnp.dot(x, wi_ref[...], preferred_element_type=f32)
    yj = jnp.dot(x, wj_ref[...], preferred_element_type=f32)
    for q in range(4):
        outs[q][...] = yi[:, q * CHQ:(q + 1) * CHQ]
        outs[4 + q][...] = yj[:, q * CHQ:(q + 1) * CHQ]


def _make_tabs(pxf, wi, wj):
    R = 600
    nb = pxf.shape[0] // R
    outs = [jax.ShapeDtypeStruct((pxf.shape[0], CHQ), f32)] * 8
    return pl.pallas_call(
        _tabs_body,
        grid=(nb,),
        in_specs=[
            pl.BlockSpec((R, C), lambda i: (i, 0)),
            pl.BlockSpec((C, C), lambda i: (0, 0)),
            pl.BlockSpec((C, C), lambda i: (0, 0)),
        ],
        out_specs=[pl.BlockSpec((R, CHQ), lambda i: (i, 0))] * 8,
        out_shape=outs,
    )(pxf, wi, wj)


def _dot_body(pxn_ref, wi_ref, wj_ref, out_ref):
    acc = None
    for x in range(3):
        v = pxn_ref[x]
        a = jnp.dot(v, wi_ref[...], preferred_element_type=f32)
        b = jnp.dot(v, wj_ref[...], preferred_element_type=f32)
        acc = a * b if acc is None else acc + a * b
    out_ref[...] = acc


def _make_dot(pxn3, wi, wj):
    Rn = 200
    grid = (N // Rn,)
    return pl.pallas_call(
        _dot_body,
        grid=grid,
        in_specs=[
            pl.BlockSpec((3, Rn, C), lambda i: (0, i, 0)),
            pl.BlockSpec((C, C), lambda i: (0, 0)),
            pl.BlockSpec((C, C), lambda i: (0, 0)),
        ],
        out_specs=pl.BlockSpec((Rn, C), lambda i: (i, 0)),
        out_shape=jax.ShapeDtypeStruct((N, C), f32),
    )(pxn3, wi, wj)


def _sc_body(indi_hbm, indj_hbm, i1_hbm, d0_hbm, d1_hbm, d2_hbm,
             ti0, ti1, ti2, ti3, tj0, tj1, tj2, tj3,
             ix_hbm, pxnew_hbm,
             idxi_v, idxj_v, gi_v, gj_v, i1_v, diff_v, ixb_v, zb_v,
             acc_sh, sem_in0, sem_in1):
    cid = lax.axis_index("c")
    sid = lax.axis_index("s")
    r0 = sid * RPT
    sem_in = (sem_in0, sem_in1)
    d_hbm = (d0_hbm, d1_hbm, d2_hbm)

    def zrow(r, _):
        for x in range(3):
            for qq in range(CHQ // 16):
                zb_v[r, x, pl.ds(qq * 16, 16)] = jnp.zeros((16,), f32)
        return 0
    lax.fori_loop(0, ZR, zrow, 0)

    pltpu.sync_copy(indi_hbm.at[sid], idxi_v)
    pltpu.sync_copy(indj_hbm.at[sid], idxj_v)

    def make_edge_body(s):
        def edge_body(e, _):
            for x in range(3):
                dv = plsc.load_gather(
                    diff_v,
                    [jnp.full((16,), s, i32), jnp.full((16,), x, i32),
                     jnp.full((16,), e, i32)])
                for qq in range(CHQ // 16):
                    sl = pl.ds(qq * 16, 16)
                    g = gi_v[s, e, x, sl] + gj_v[s, e, x, sl] + dv
                    ixb_v[s, e, x, sl] = g * i1_v[s, e, sl]
            return 0
        return edge_body

    edge_bodies = (make_edge_body(0), make_edge_body(1))

    def in_copies(t, s, tab_i, tab_j, coff):
        base = sid * EPT + t * K
        return (
            (tab_i.at[idxi_v.at[t]], gi_v.at[s], sem_in[s]),
            (tab_j.at[idxj_v.at[t]], gj_v.at[s], sem_in[s]),
            (i1_hbm.at[sid, t, :, pl.ds(coff, CHQ)], i1_v.at[s], sem_in[s]),
            (d_hbm[0].at[pl.ds(base, K)], diff_v.at[s, 0], sem_in[s]),
            (d_hbm[1].at[pl.ds(base, K)], diff_v.at[s, 1], sem_in[s]),
            (d_hbm[2].at[pl.ds(base, K)], diff_v.at[s, 2], sem_in[s]),
        )

    def issue(cs):
        for src, dst, sem in cs:
            pltpu.async_copy(src, dst, sem)

    def drain(cs):
        for src, dst, sem in cs:
            pltpu.make_async_copy(src, dst, sem).wait()

    def run_pass(tab_i, tab_j, coff):
        issue(in_copies(0, 0, tab_i, tab_j, coff))

        def body2(tt, _):
            t0 = 2 * tt
            for s in range(2):
                t = t0 + s
                drain(in_copies(t, s, tab_i, tab_j, coff))
                nxt = t + 1

                @pl.when(nxt < NCHK)
                def _():
                    issue(in_copies(nxt, 1 - s, tab_i, tab_j, coff))

                lax.fori_loop(0, K, edge_bodies[s], 0)
                for x in range(3):
                    pltpu.sync_copy(
                        ixb_v.at[s, :, x, :],
                        ix_hbm.at[x, sid, t, :, pl.ds(coff, CHQ)])
                pltpu.sync_copy(ixb_v.at[s], acc_sh.at[idxi_v.at[t]],
                                add=True)
            return 0

        lax.fori_loop(0, NCHK // 2, body2, 0)

    for h in range(2):
        for z in range(RPT // ZR):
            pltpu.sync_copy(zb_v, acc_sh.at[pl.ds(r0 + z * ZR, ZR)])
        plsc.subcore_barrier()

        tabs = ((ti0, tj0, 0), (ti1, tj1, CHQ)) if h == 0 else \
               ((ti2, tj2, 2 * CHQ), (ti3, tj3, 3 * CHQ))

        @pl.when(cid == 0)
        def _():
            run_pass(tabs[0][0], tabs[0][1], tabs[0][2])

        @pl.when(cid == 1)
        def _():
            run_pass(tabs[1][0], tabs[1][1], tabs[1][2])

        plsc.subcore_barrier()

        for x in range(3):
            @pl.when(cid == 0)
            def _():
                pltpu.sync_copy(
                    acc_sh.at[pl.ds(r0, RPT), x, :],
                    pxnew_hbm.at[x, pl.ds(r0, RPT), pl.ds(tabs[0][2], CHQ)])

            @pl.when(cid == 1)
            def _():
                pltpu.sync_copy(
                    acc_sh.at[pl.ds(r0, RPT), x, :],
                    pxnew_hbm.at[x, pl.ds(r0, RPT), pl.ds(tabs[1][2], CHQ)])


def _run_sc(indi, indj, i1r, d0, d1, d2, tabs):
    mesh = plsc.VectorSubcoreMesh(core_axis_name="c", subcore_axis_name="s")
    k = pl.kernel(
        _sc_body,
        out_type=[
            jax.ShapeDtypeStruct((3, NS, NCHK, K, C), f32),
            jax.ShapeDtypeStruct((3, N, C), f32),
        ],
        mesh=mesh,
        compiler_params=pltpu.CompilerParams(use_tc_tiling_on_sc=False,
                                             needs_layout_passes=False),
        scratch_types=[
            pltpu.VMEM((NCHK, K), i32),
            pltpu.VMEM((NCHK, K), i32),
            pltpu.VMEM((2, K, 3, CHQ), f32),
            pltpu.VMEM((2, K, 3, CHQ), f32),
            pltpu.VMEM((2, K, CHQ), f32),
            pltpu.VMEM((2, 3, K), f32),
            pltpu.VMEM((2, K, 3, CHQ), f32),
            pltpu.VMEM((ZR, 3, CHQ), f32),
            pltpu.VMEM_SHARED((N, 3, CHQ), f32),
            pltpu.SemaphoreType.DMA,
            pltpu.SemaphoreType.DMA,
        ],
    )
    return k(indi, indj, i1r, d0, d1, d2, *tabs)


def kernel(ind_2, px, i1, diff, W_pi_i, W_pi_j, W_dot_i, W_dot_j):
    ind_i = ind_2[:, 0].reshape(NS, NCHK, K)
    ind_j = ind_2[:, 1].reshape(NS, NCHK, K)
    i1r = i1.reshape(NS, NCHK, K, C)
    d0, d1, d2 = diff[:, 0], diff[:, 1], diff[:, 2]

    pxf = px.reshape(N * 3, C)
    tabs = _make_tabs(pxf, W_pi_i, W_pi_j)
    tabs = [t.reshape(N, 3, CHQ) for t in tabs]

    ix3, pxn3 = _run_sc(ind_i, ind_j, i1r, d0, d1, d2, tabs)
    ix = jnp.transpose(ix3.reshape(3, E, C), (1, 0, 2))
    px_new = jnp.transpose(pxn3, (1, 0, 2))

    dotted = _make_dot(pxn3, W_dot_i, W_dot_j)
    return px_new, ix, dotted


# speedup vs baseline: 11.6961x; 1.0466x over previous
import jax
import jax.numpy as jnp
from jax import lax
from jax.experimental import pallas as pl
from jax.experimental.pallas import tpu as pltpu
from jax.experimental.pallas import tpu_sc as plsc

N = 10000
E = 160000
C = 128
CHQ = 32
NC = 2
NS = 16
K = 40
EPT = E // NS
NCHK = EPT // K
RPT = N // NS
ZR = 25

f32 = jnp.float32
i32 = jnp.int32


def _tabs_body(px_ref, wi_ref, wj_ref, *outs):
    x = px_ref[...]
    R = x.shape[0]
    yi = jnp.dot(x, wi_ref[...], preferred_element_type=f32)
    yj = jnp.dot(x, wj_ref[...], preferred_element_type=f32)
    for q in range(4):
        outs[q][...] = yi[:, q * CHQ:(q + 1) * CHQ]
        outs[4 + q][...] = yj[:, q * CHQ:(q + 1) * CHQ]


def _make_tabs(pxf, wi, wj):
    R = 600
    nb = pxf.shape[0] // R
    outs = [jax.ShapeDtypeStruct((pxf.shape[0], CHQ), f32)] * 8
    return pl.pallas_call(
        _tabs_body,
        grid=(nb,),
        in_specs=[
            pl.BlockSpec((R, C), lambda i: (i, 0)),
            pl.BlockSpec((C, C), lambda i: (0, 0)),
            pl.BlockSpec((C, C), lambda i: (0, 0)),
        ],
        out_specs=[pl.BlockSpec((R, CHQ), lambda i: (i, 0))] * 8,
        out_shape=outs,
    )(pxf, wi, wj)


def _dot_body(pxn_ref, wi_ref, wj_ref, out_ref):
    acc = None
    for x in range(3):
        v = pxn_ref[x]
        a = jnp.dot(v, wi_ref[...], preferred_element_type=f32)
        b = jnp.dot(v, wj_ref[...], preferred_element_type=f32)
        acc = a * b if acc is None else acc + a * b
    out_ref[...] = acc


def _make_dot(pxn3, wi, wj):
    Rn = 200
    grid = (N // Rn,)
    return pl.pallas_call(
        _dot_body,
        grid=grid,
        in_specs=[
            pl.BlockSpec((3, Rn, C), lambda i: (0, i, 0)),
            pl.BlockSpec((C, C), lambda i: (0, 0)),
            pl.BlockSpec((C, C), lambda i: (0, 0)),
        ],
        out_specs=pl.BlockSpec((Rn, C), lambda i: (i, 0)),
        out_shape=jax.ShapeDtypeStruct((N, C), f32),
    )(pxn3, wi, wj)


def _sc_body(indi_hbm, indj_hbm, i1_hbm, d0_hbm, d1_hbm, d2_hbm,
             ti0, ti1, ti2, ti3, tj0, tj1, tj2, tj3,
             ix_hbm, pxnew_hbm,
             idxi_v, idxj_v, gi_v, gj_v, i1_v, diff_v, ixb_v, zb_v,
             acc_sh, sem_in0, sem_in1, sem_out0, sem_out1):
    cid = lax.axis_index("c")
    sid = lax.axis_index("s")
    r0 = sid * RPT
    sem_in = (sem_in0, sem_in1)
    sem_out = (sem_out0, sem_out1)
    d_hbm = (d0_hbm, d1_hbm, d2_hbm)

    def zrow(r, _):
        for x in range(3):
            for qq in range(CHQ // 16):
                zb_v[r, x, pl.ds(qq * 16, 16)] = jnp.zeros((16,), f32)
        return 0
    lax.fori_loop(0, ZR, zrow, 0)

    pltpu.sync_copy(indi_hbm.at[sid], idxi_v)
    pltpu.sync_copy(indj_hbm.at[sid], idxj_v)

    def make_edge_body(s):
        def edge_body(e, _):
            for x in range(3):
                dv = plsc.load_gather(
                    diff_v,
                    [jnp.full((16,), s, i32), jnp.full((16,), x, i32),
                     jnp.full((16,), e, i32)])
                for qq in range(CHQ // 16):
                    sl = pl.ds(qq * 16, 16)
                    g = gi_v[s, e, x, sl] + gj_v[s, e, x, sl] + dv
                    ixb_v[s, e, x, sl] = g * i1_v[s, e, sl]
            return 0
        return edge_body

    edge_bodies = (make_edge_body(0), make_edge_body(1))

    def in_copies(t, s, tab_i, tab_j, coff):
        base = sid * EPT + t * K
        return (
            (tab_i.at[idxi_v.at[t]], gi_v.at[s], sem_in[s]),
            (tab_j.at[idxj_v.at[t]], gj_v.at[s], sem_in[s]),
            (i1_hbm.at[sid, t, :, pl.ds(coff, CHQ)], i1_v.at[s], sem_in[s]),
            (d_hbm[0].at[pl.ds(base, K)], diff_v.at[s, 0], sem_in[s]),
            (d_hbm[1].at[pl.ds(base, K)], diff_v.at[s, 1], sem_in[s]),
            (d_hbm[2].at[pl.ds(base, K)], diff_v.at[s, 2], sem_in[s]),
        )

    def out_copies(t, s, coff):
        return tuple(
            (ixb_v.at[s, :, x, :],
             ix_hbm.at[x, sid, t, :, pl.ds(coff, CHQ)], sem_out[s])
            for x in range(3))

    def issue(cs):
        for src, dst, sem in cs:
            pltpu.async_copy(src, dst, sem)

    def drain(cs):
        for src, dst, sem in cs:
            pltpu.make_async_copy(src, dst, sem).wait()

    def run_pass(tab_i, tab_j, coff):
        issue(in_copies(0, 0, tab_i, tab_j, coff))

        def body2(tt, _):
            t0 = 2 * tt
            for s in range(2):
                t = t0 + s
                drain(in_copies(t, s, tab_i, tab_j, coff))
                nxt = t + 1

                @pl.when(nxt < NCHK)
                def _():
                    issue(in_copies(nxt, 1 - s, tab_i, tab_j, coff))

                @pl.when(tt > 0)
                def _():
                    drain(out_copies(t - 2, s, coff))

                lax.fori_loop(0, K, edge_bodies[s], 0)
                issue(out_copies(t, s, coff))
                pltpu.sync_copy(ixb_v.at[s], acc_sh.at[idxi_v.at[t]],
                                add=True)
            return 0

        lax.fori_loop(0, NCHK // 2, body2, 0)
        drain(out_copies(NCHK - 2, 0, coff))
        drain(out_copies(NCHK - 1, 1, coff))

    for h in range(2):
        for z in range(RPT // ZR):
            pltpu.sync_copy(zb_v, acc_sh.at[pl.ds(r0 + z * ZR, ZR)])
        plsc.subcore_barrier()

        tabs = ((ti0, tj0, 0), (ti1, tj1, CHQ)) if h == 0 else \
               ((ti2, tj2, 2 * CHQ), (ti3, tj3, 3 * CHQ))

        @pl.when(cid == 0)
        def _():
            run_pass(tabs[0][0], tabs[0][1], tabs[0][2])

        @pl.when(cid == 1)
        def _():
            run_pass(tabs[1][0], tabs[1][1], tabs[1][2])

        plsc.subcore_barrier()

        for x in range(3):
            @pl.when(cid == 0)
            def _():
                pltpu.sync_copy(
                    acc_sh.at[pl.ds(r0, RPT), x, :],
                    pxnew_hbm.at[x, pl.ds(r0, RPT), pl.ds(tabs[0][2], CHQ)])

            @pl.when(cid == 1)
            def _():
                pltpu.sync_copy(
                    acc_sh.at[pl.ds(r0, RPT), x, :],
                    pxnew_hbm.at[x, pl.ds(r0, RPT), pl.ds(tabs[1][2], CHQ)])


def _run_sc(indi, indj, i1r, d0, d1, d2, tabs):
    mesh = plsc.VectorSubcoreMesh(core_axis_name="c", subcore_axis_name="s")
    k = pl.kernel(
        _sc_body,
        out_type=[
            jax.ShapeDtypeStruct((3, NS, NCHK, K, C), f32),
            jax.ShapeDtypeStruct((3, N, C), f32),
        ],
        mesh=mesh,
        compiler_params=pltpu.CompilerParams(use_tc_tiling_on_sc=False,
                                             needs_layout_passes=False),
        scratch_types=[
            pltpu.VMEM((NCHK, K), i32),
            pltpu.VMEM((NCHK, K), i32),
            pltpu.VMEM((2, K, 3, CHQ), f32),
            pltpu.VMEM((2, K, 3, CHQ), f32),
            pltpu.VMEM((2, K, CHQ), f32),
            pltpu.VMEM((2, 3, K), f32),
            pltpu.VMEM((2, K, 3, CHQ), f32),
            pltpu.VMEM((ZR, 3, CHQ), f32),
            pltpu.VMEM_SHARED((N, 3, CHQ), f32),
            pltpu.SemaphoreType.DMA,
            pltpu.SemaphoreType.DMA,
            pltpu.SemaphoreType.DMA,
            pltpu.SemaphoreType.DMA,
        ],
    )
    return k(indi, indj, i1r, d0, d1, d2, *tabs)


def kernel(ind_2, px, i1, diff, W_pi_i, W_pi_j, W_dot_i, W_dot_j):
    ind_i = ind_2[:, 0].reshape(NS, NCHK, K)
    ind_j = ind_2[:, 1].reshape(NS, NCHK, K)
    i1r = i1.reshape(NS, NCHK, K, C)
    d0, d1, d2 = diff[:, 0], diff[:, 1], diff[:, 2]

    pxf = px.reshape(N * 3, C)
    tabs = _make_tabs(pxf, W_pi_i, W_pi_j)
    tabs = [t.reshape(N, 3, CHQ) for t in tabs]

    ix3, pxn3 = _run_sc(ind_i, ind_j, i1r, d0, d1, d2, tabs)
    ix = jnp.transpose(ix3.reshape(3, E, C), (1, 0, 2))
    px_new = jnp.transpose(pxn3, (1, 0, 2))

    dotted = _make_dot(pxn3, W_dot_i, W_dot_j)
    return px_new, ix, dotted


# speedup vs baseline: 20.5893x; 1.7604x over previous
import jax
import jax.numpy as jnp
from jax import lax
from jax.experimental import pallas as pl
from jax.experimental.pallas import tpu as pltpu
from jax.experimental.pallas import tpu_sc as plsc

N = 10000
E = 160000
C = 128
CHQ = 32
NC = 2
NS = 16
K = 40
EPT = E // NS
NCHK = EPT // K
RPT = N // NS
ZR = 25

f32 = jnp.float32
i32 = jnp.int32


def _tabs_body(px_ref, wi_ref, wj_ref, *outs):
    x = px_ref[...]
    R = x.shape[0]
    yi = jnp.dot(x, wi_ref[...], preferred_element_type=f32)
    yj = jnp.dot(x, wj_ref[...], preferred_element_type=f32)
    for q in range(4):
        outs[q][...] = yi[:, q * CHQ:(q + 1) * CHQ]
        outs[4 + q][...] = yj[:, q * CHQ:(q + 1) * CHQ]


def _make_tabs(pxf, wi, wj):
    R = 600
    nb = pxf.shape[0] // R
    outs = [jax.ShapeDtypeStruct((pxf.shape[0], CHQ), f32)] * 8
    return pl.pallas_call(
        _tabs_body,
        grid=(nb,),
        in_specs=[
            pl.BlockSpec((R, C), lambda i: (i, 0)),
            pl.BlockSpec((C, C), lambda i: (0, 0)),
            pl.BlockSpec((C, C), lambda i: (0, 0)),
        ],
        out_specs=[pl.BlockSpec((R, CHQ), lambda i: (i, 0))] * 8,
        out_shape=outs,
    )(pxf, wi, wj)


def _dot_body(pxn_ref, wi_ref, wj_ref, out_ref):
    acc = None
    for x in range(3):
        v = pxn_ref[x]
        a = jnp.dot(v, wi_ref[...], preferred_element_type=f32)
        b = jnp.dot(v, wj_ref[...], preferred_element_type=f32)
        acc = a * b if acc is None else acc + a * b
    out_ref[...] = acc


def _make_dot(pxn3, wi, wj):
    Rn = 200
    grid = (N // Rn,)
    return pl.pallas_call(
        _dot_body,
        grid=grid,
        in_specs=[
            pl.BlockSpec((3, Rn, C), lambda i: (0, i, 0)),
            pl.BlockSpec((C, C), lambda i: (0, 0)),
            pl.BlockSpec((C, C), lambda i: (0, 0)),
        ],
        out_specs=pl.BlockSpec((Rn, C), lambda i: (i, 0)),
        out_shape=jax.ShapeDtypeStruct((N, C), f32),
    )(pxn3, wi, wj)


def _sc_body(indi_hbm, indj_hbm, i1_hbm, d0_hbm, d1_hbm, d2_hbm,
             ti0, ti1, ti2, ti3, tj0, tj1, tj2, tj3,
             ix_hbm, pxnew_hbm,
             idxi_v, idxj_v, gi_v, gj_v, i1_v, diff_v, ixb_v, zb_v,
             acc_sh, sem_in0, sem_in1, sem_out0, sem_out1):
    cid = lax.axis_index("c")
    sid = lax.axis_index("s")
    r0 = sid * RPT
    sem_in = (sem_in0, sem_in1)
    sem_out = (sem_out0, sem_out1)
    d_hbm = (d0_hbm, d1_hbm, d2_hbm)

    def zrow(r, _):
        for x in range(3):
            for qq in range(CHQ // 16):
                zb_v[r, x, pl.ds(qq * 16, 16)] = jnp.zeros((16,), f32)
        return 0
    lax.fori_loop(0, ZR, zrow, 0)

    pltpu.sync_copy(indi_hbm.at[sid], idxi_v)
    pltpu.sync_copy(indj_hbm.at[sid], idxj_v)

    def make_edge_body(s):
        def edge_body(e):
            i1lo = i1_v[s, e, pl.ds(0, 16)]
            i1hi = i1_v[s, e, pl.ds(16, 16)]
            ev = jnp.full((16,), e, i32)
            for x in range(3):
                dv = plsc.load_gather(
                    diff_v,
                    [jnp.full((16,), s, i32), jnp.full((16,), x, i32), ev])
                lo = pl.ds(0, 16)
                hi = pl.ds(16, 16)
                ixb_v[s, e, x, lo] = (
                    gi_v[s, e, x, lo] + gj_v[s, e, x, lo] + dv) * i1lo
                ixb_v[s, e, x, hi] = (
                    gi_v[s, e, x, hi] + gj_v[s, e, x, hi] + dv) * i1hi
        return edge_body

    edge_bodies = (make_edge_body(0), make_edge_body(1))

    def in_copies(t, s, tab_i, tab_j, coff):
        base = sid * EPT + t * K
        return (
            (tab_i.at[idxi_v.at[t]], gi_v.at[s], sem_in[s]),
            (tab_j.at[idxj_v.at[t]], gj_v.at[s], sem_in[s]),
            (i1_hbm.at[sid, t, :, pl.ds(coff, CHQ)], i1_v.at[s], sem_in[s]),
            (d_hbm[0].at[pl.ds(base, K)], diff_v.at[s, 0], sem_in[s]),
            (d_hbm[1].at[pl.ds(base, K)], diff_v.at[s, 1], sem_in[s]),
            (d_hbm[2].at[pl.ds(base, K)], diff_v.at[s, 2], sem_in[s]),
        )

    def out_copies(t, s, coff):
        return tuple(
            (ixb_v.at[s, :, x, :],
             ix_hbm.at[x, sid, t, :, pl.ds(coff, CHQ)], sem_out[s])
            for x in range(3))


    def issue(cs):
        for src, dst, sem in cs:
            pltpu.async_copy(src, dst, sem)

    def drain(cs):
        for src, dst, sem in cs:
            pltpu.make_async_copy(src, dst, sem).wait()

    def run_pass(tab_i, tab_j, coff):
        issue(in_copies(0, 0, tab_i, tab_j, coff))

        def body2(tt, _):
            t0 = 2 * tt
            for s in range(2):
                t = t0 + s
                drain(in_copies(t, s, tab_i, tab_j, coff))
                nxt = t + 1

                @pl.when(nxt < NCHK)
                def _():
                    issue(in_copies(nxt, 1 - s, tab_i, tab_j, coff))

                @pl.when(tt > 0)
                def _():
                    drain(out_copies(t - 2, s, coff))

                plsc.parallel_loop(0, K, 1, unroll=4)(edge_bodies[s])
                issue(out_copies(t, s, coff))
                pltpu.sync_copy(ixb_v.at[s], acc_sh.at[idxi_v.at[t]],
                                add=True)
            return 0

        lax.fori_loop(0, NCHK // 2, body2, 0)
        drain(out_copies(NCHK - 2, 0, coff))
        drain(out_copies(NCHK - 1, 1, coff))

    for h in range(2):
        for z in range(RPT // ZR):
            pltpu.sync_copy(zb_v, acc_sh.at[pl.ds(r0 + z * ZR, ZR)])
        plsc.subcore_barrier()

        tabs = ((ti0, tj0, 0), (ti1, tj1, CHQ)) if h == 0 else \
               ((ti2, tj2, 2 * CHQ), (ti3, tj3, 3 * CHQ))

        @pl.when(cid == 0)
        def _():
            run_pass(tabs[0][0], tabs[0][1], tabs[0][2])

        @pl.when(cid == 1)
        def _():
            run_pass(tabs[1][0], tabs[1][1], tabs[1][2])

        plsc.subcore_barrier()

        for x in range(3):
            @pl.when(cid == 0)
            def _():
                pltpu.sync_copy(
                    acc_sh.at[pl.ds(r0, RPT), x, :],
                    pxnew_hbm.at[x, pl.ds(r0, RPT), pl.ds(tabs[0][2], CHQ)])

            @pl.when(cid == 1)
            def _():
                pltpu.sync_copy(
                    acc_sh.at[pl.ds(r0, RPT), x, :],
                    pxnew_hbm.at[x, pl.ds(r0, RPT), pl.ds(tabs[1][2], CHQ)])


def _run_sc(indi, indj, i1r, d0, d1, d2, tabs):
    mesh = plsc.VectorSubcoreMesh(core_axis_name="c", subcore_axis_name="s")
    k = pl.kernel(
        _sc_body,
        out_type=[
            jax.ShapeDtypeStruct((3, NS, NCHK, K, C), f32),
            jax.ShapeDtypeStruct((3, N, C), f32),
        ],
        mesh=mesh,
        compiler_params=pltpu.CompilerParams(use_tc_tiling_on_sc=False,
                                             needs_layout_passes=False),
        scratch_types=[
            pltpu.VMEM((NCHK, K), i32),
            pltpu.VMEM((NCHK, K), i32),
            pltpu.VMEM((2, K, 3, CHQ), f32),
            pltpu.VMEM((2, K, 3, CHQ), f32),
            pltpu.VMEM((2, K, CHQ), f32),
            pltpu.VMEM((2, 3, K), f32),
            pltpu.VMEM((2, K, 3, CHQ), f32),
            pltpu.VMEM((ZR, 3, CHQ), f32),
            pltpu.VMEM_SHARED((N, 3, CHQ), f32),
            pltpu.SemaphoreType.DMA,
            pltpu.SemaphoreType.DMA,
            pltpu.SemaphoreType.DMA,
            pltpu.SemaphoreType.DMA,
        ],
    )
    return k(indi, indj, i1r, d0, d1, d2, *tabs)


def kernel(ind_2, px, i1, diff, W_pi_i, W_pi_j, W_dot_i, W_dot_j):
    ind_i = ind_2[:, 0].reshape(NS, NCHK, K)
    ind_j = ind_2[:, 1].reshape(NS, NCHK, K)
    i1r = i1.reshape(NS, NCHK, K, C)
    d0, d1, d2 = diff[:, 0], diff[:, 1], diff[:, 2]

    pxf = px.reshape(N * 3, C)
    tabs = _make_tabs(pxf, W_pi_i, W_pi_j)
    tabs = [t.reshape(N, 3, CHQ) for t in tabs]

    ix3, pxn3 = _run_sc(ind_i, ind_j, i1r, d0, d1, d2, tabs)
    ix = jnp.transpose(ix3.reshape(3, E, C), (1, 0, 2))
    px_new = jnp.transpose(pxn3, (1, 0, 2))

    dotted = _make_dot(pxn3, W_dot_i, W_dot_j)
    return px_new, ix, dotted


# speedup vs baseline: 22.0059x; 1.0688x over previous
import jax
import jax.numpy as jnp
from jax import lax
from jax.experimental import pallas as pl
from jax.experimental.pallas import tpu as pltpu
from jax.experimental.pallas import tpu_sc as plsc

N = 10000
E = 160000
C = 128
CHQ = 32
NC = 2
NS = 16
K = 40
EPT = E // NS
NCHK = EPT // K
RPT = N // NS
ZR = 25

f32 = jnp.float32
i32 = jnp.int32


NPF = 30720
TROWS = NPF // 4


def _tabs_body(px4_ref, *refs):
    wbds = refs[:8]
    outs = refs[8:]
    x4 = px4_ref[...]
    for q in range(8):
        outs[q][...] = jnp.dot(x4, wbds[q][...], preferred_element_type=f32)


def _make_tabs(pxf4, wbds):
    R4 = 256
    nb = pxf4.shape[0] // R4
    outs = [jax.ShapeDtypeStruct((TROWS, C), f32)] * 8
    return pl.pallas_call(
        _tabs_body,
        grid=(nb,),
        in_specs=[pl.BlockSpec((R4, 4 * C), lambda i: (i, 0))] +
                 [pl.BlockSpec((4 * C, C), lambda i: (0, 0))] * 8,
        out_specs=[pl.BlockSpec((R4, C), lambda i: (i, 0))] * 8,
        out_shape=outs,
    )(pxf4, *wbds)


def _block_diag_quarters(wi, wj):
    wbds = []
    for w in (wi, wj):
        for q in range(4):
            wq = w[:, q * CHQ:(q + 1) * CHQ]
            z = jnp.zeros((4 * C, C), f32)
            for u in range(4):
                z = z.at[u * C:(u + 1) * C, u * CHQ:(u + 1) * CHQ].set(wq)
            wbds.append(z)
    return wbds


def _dot_body(pxn_ref, wi_ref, wj_ref, out_ref):
    acc = None
    for x in range(3):
        v = pxn_ref[x]
        a = jnp.dot(v, wi_ref[...], preferred_element_type=f32)
        b = jnp.dot(v, wj_ref[...], preferred_element_type=f32)
        acc = a * b if acc is None else acc + a * b
    out_ref[...] = acc


def _make_dot(pxn3, wi, wj):
    Rn = 200
    grid = (N // Rn,)
    return pl.pallas_call(
        _dot_body,
        grid=grid,
        in_specs=[
            pl.BlockSpec((3, Rn, C), lambda i: (0, i, 0)),
            pl.BlockSpec((C, C), lambda i: (0, 0)),
            pl.BlockSpec((C, C), lambda i: (0, 0)),
        ],
        out_specs=pl.BlockSpec((Rn, C), lambda i: (i, 0)),
        out_shape=jax.ShapeDtypeStruct((N, C), f32),
    )(pxn3, wi, wj)


def _sc_body(indi_hbm, indj_hbm, i1_hbm, d0_hbm, d1_hbm, d2_hbm,
             ti0, ti1, ti2, ti3, tj0, tj1, tj2, tj3,
             ix_hbm, pxnew_hbm,
             idxi_v, idxj_v, gi_v, gj_v, i1_v, diff_v, ixb_v, zb_v,
             acc_sh, sem_in0, sem_in1, sem_out0, sem_out1):
    cid = lax.axis_index("c")
    sid = lax.axis_index("s")
    r0 = sid * RPT
    sem_in = (sem_in0, sem_in1)
    sem_out = (sem_out0, sem_out1)
    d_hbm = (d0_hbm, d1_hbm, d2_hbm)

    def zrow(r, _):
        for x in range(3):
            for qq in range(CHQ // 16):
                zb_v[r, x, pl.ds(qq * 16, 16)] = jnp.zeros((16,), f32)
        return 0
    lax.fori_loop(0, ZR, zrow, 0)

    pltpu.sync_copy(indi_hbm.at[sid], idxi_v)
    pltpu.sync_copy(indj_hbm.at[sid], idxj_v)

    def make_edge_body(s):
        def edge_body(e):
            i1lo = i1_v[s, e, pl.ds(0, 16)]
            i1hi = i1_v[s, e, pl.ds(16, 16)]
            ev = jnp.full((16,), e, i32)
            for x in range(3):
                dv = plsc.load_gather(
                    diff_v,
                    [jnp.full((16,), s, i32), jnp.full((16,), x, i32), ev])
                lo = pl.ds(0, 16)
                hi = pl.ds(16, 16)
                ixb_v[s, e, x, lo] = (
                    gi_v[s, e, x, lo] + gj_v[s, e, x, lo] + dv) * i1lo
                ixb_v[s, e, x, hi] = (
                    gi_v[s, e, x, hi] + gj_v[s, e, x, hi] + dv) * i1hi
        return edge_body

    edge_bodies = (make_edge_body(0), make_edge_body(1))

    def in_copies(t, s, tab_i, tab_j, coff):
        base = sid * EPT + t * K
        return (
            (tab_i.at[idxi_v.at[t]], gi_v.at[s], sem_in[s]),
            (tab_j.at[idxj_v.at[t]], gj_v.at[s], sem_in[s]),
            (i1_hbm.at[sid, t, :, pl.ds(coff, CHQ)], i1_v.at[s], sem_in[s]),
            (d_hbm[0].at[pl.ds(base, K)], diff_v.at[s, 0], sem_in[s]),
            (d_hbm[1].at[pl.ds(base, K)], diff_v.at[s, 1], sem_in[s]),
            (d_hbm[2].at[pl.ds(base, K)], diff_v.at[s, 2], sem_in[s]),
        )

    def out_copies(t, s, coff):
        return tuple(
            (ixb_v.at[s, :, x, :],
             ix_hbm.at[x, sid, t, :, pl.ds(coff, CHQ)], sem_out[s])
            for x in range(3))


    def issue(cs):
        for src, dst, sem in cs:
            pltpu.async_copy(src, dst, sem)

    def drain(cs):
        for src, dst, sem in cs:
            pltpu.make_async_copy(src, dst, sem).wait()

    def run_pass(tab_i, tab_j, coff):
        issue(in_copies(0, 0, tab_i, tab_j, coff))

        def body2(tt, _):
            t0 = 2 * tt
            for s in range(2):
                t = t0 + s
                drain(in_copies(t, s, tab_i, tab_j, coff))
                nxt = t + 1

                @pl.when(nxt < NCHK)
                def _():
                    issue(in_copies(nxt, 1 - s, tab_i, tab_j, coff))

                @pl.when(tt > 0)
                def _():
                    drain(out_copies(t - 2, s, coff))

                plsc.parallel_loop(0, K, 1, unroll=4)(edge_bodies[s])
                issue(out_copies(t, s, coff))
                pltpu.sync_copy(ixb_v.at[s], acc_sh.at[idxi_v.at[t]],
                                add=True)
            return 0

        lax.fori_loop(0, NCHK // 2, body2, 0)
        drain(out_copies(NCHK - 2, 0, coff))
        drain(out_copies(NCHK - 1, 1, coff))

    for h in range(2):
        for z in range(RPT // ZR):
            pltpu.sync_copy(zb_v, acc_sh.at[pl.ds(r0 + z * ZR, ZR)])
        plsc.subcore_barrier()

        tabs = ((ti0, tj0, 0), (ti1, tj1, CHQ)) if h == 0 else \
               ((ti2, tj2, 2 * CHQ), (ti3, tj3, 3 * CHQ))

        @pl.when(cid == 0)
        def _():
            run_pass(tabs[0][0], tabs[0][1], tabs[0][2])

        @pl.when(cid == 1)
        def _():
            run_pass(tabs[1][0], tabs[1][1], tabs[1][2])

        plsc.subcore_barrier()

        for x in range(3):
            @pl.when(cid == 0)
            def _():
                pltpu.sync_copy(
                    acc_sh.at[pl.ds(r0, RPT), x, :],
                    pxnew_hbm.at[x, pl.ds(r0, RPT), pl.ds(tabs[0][2], CHQ)])

            @pl.when(cid == 1)
            def _():
                pltpu.sync_copy(
                    acc_sh.at[pl.ds(r0, RPT), x, :],
                    pxnew_hbm.at[x, pl.ds(r0, RPT), pl.ds(tabs[1][2], CHQ)])


def _run_sc(indi, indj, i1r, d0, d1, d2, tabs):
    mesh = plsc.VectorSubcoreMesh(core_axis_name="c", subcore_axis_name="s")
    k = pl.kernel(
        _sc_body,
        out_type=[
            jax.ShapeDtypeStruct((3, NS, NCHK, K, C), f32),
            jax.ShapeDtypeStruct((3, N, C), f32),
        ],
        mesh=mesh,
        compiler_params=pltpu.CompilerParams(use_tc_tiling_on_sc=False,
                                             needs_layout_passes=False),
        scratch_types=[
            pltpu.VMEM((NCHK, K), i32),
            pltpu.VMEM((NCHK, K), i32),
            pltpu.VMEM((2, K, 3, CHQ), f32),
            pltpu.VMEM((2, K, 3, CHQ), f32),
            pltpu.VMEM((2, K, CHQ), f32),
            pltpu.VMEM((2, 3, K), f32),
            pltpu.VMEM((2, K, 3, CHQ), f32),
            pltpu.VMEM((ZR, 3, CHQ), f32),
            pltpu.VMEM_SHARED((N, 3, CHQ), f32),
            pltpu.SemaphoreType.DMA,
            pltpu.SemaphoreType.DMA,
            pltpu.SemaphoreType.DMA,
            pltpu.SemaphoreType.DMA,
        ],
    )
    return k(indi, indj, i1r, d0, d1, d2, *tabs)


def kernel(ind_2, px, i1, diff, W_pi_i, W_pi_j, W_dot_i, W_dot_j):
    ind_i = ind_2[:, 0].reshape(NS, NCHK, K)
    ind_j = ind_2[:, 1].reshape(NS, NCHK, K)
    i1r = i1.reshape(NS, NCHK, K, C)
    d0, d1, d2 = diff[:, 0], diff[:, 1], diff[:, 2]

    pxf4 = jnp.pad(px.reshape(N * 3, C),
                   ((0, NPF - N * 3), (0, 0))).reshape(NPF // 4, 4 * C)
    wbds = _block_diag_quarters(W_pi_i, W_pi_j)
    tabs = _make_tabs(pxf4, wbds)
    tabs = [t.reshape(TROWS * C // (3 * CHQ), 3, CHQ) for t in tabs]

    ix3, pxn3 = _run_sc(ind_i, ind_j, i1r, d0, d1, d2, tabs)
    ix = jnp.transpose(ix3.reshape(3, E, C), (1, 0, 2))
    px_new = jnp.transpose(pxn3, (1, 0, 2))

    dotted = _make_dot(pxn3, W_dot_i, W_dot_j)
    return px_new, ix, dotted


# speedup vs baseline: 22.2412x; 1.0107x over previous
import jax
import jax.numpy as jnp
from jax import lax
from jax.experimental import pallas as pl
from jax.experimental.pallas import tpu as pltpu
from jax.experimental.pallas import tpu_sc as plsc

N = 10000
E = 160000
C = 128
CHQ = 32
NC = 2
NS = 16
K = 40
EPT = E // NS
NCHK = EPT // K
RPT = N // NS
ZR = 25

f32 = jnp.float32
i32 = jnp.int32


NPF = 30720
TROWS = NPF // 4


def _tabs_body(px4_ref, *refs):
    wbds = refs[:8]
    outs = refs[8:]
    x4 = px4_ref[...]
    for q in range(8):
        outs[q][...] = jnp.dot(
            x4, wbds[q][...],
            preferred_element_type=f32).astype(jnp.bfloat16)


def _make_tabs(pxf4, wbds):
    R4 = 256
    nb = pxf4.shape[0] // R4
    outs = [jax.ShapeDtypeStruct((TROWS, C), jnp.bfloat16)] * 8
    return pl.pallas_call(
        _tabs_body,
        grid=(nb,),
        in_specs=[pl.BlockSpec((R4, 4 * C), lambda i: (i, 0))] +
                 [pl.BlockSpec((4 * C, C), lambda i: (0, 0))] * 8,
        out_specs=[pl.BlockSpec((R4, C), lambda i: (i, 0))] * 8,
        out_shape=outs,
    )(pxf4, *wbds)


def _block_diag_quarters(wi, wj):
    perm = jnp.array([v for i in range(16) for v in (i, 16 + i)], i32)
    wbds = []
    for w in (wi, wj):
        for q in range(4):
            wq = w[:, q * CHQ:(q + 1) * CHQ][:, perm]
            z = jnp.zeros((4 * C, C), f32)
            for u in range(4):
                z = z.at[u * C:(u + 1) * C, u * CHQ:(u + 1) * CHQ].set(wq)
            wbds.append(z)
    return wbds


def _dot_body(pxn_ref, wi_ref, wj_ref, out_ref):
    acc = None
    for x in range(3):
        v = pxn_ref[x]
        a = jnp.dot(v, wi_ref[...], preferred_element_type=f32)
        b = jnp.dot(v, wj_ref[...], preferred_element_type=f32)
        acc = a * b if acc is None else acc + a * b
    out_ref[...] = acc


def _make_dot(pxn3, wi, wj):
    Rn = 200
    grid = (N // Rn,)
    return pl.pallas_call(
        _dot_body,
        grid=grid,
        in_specs=[
            pl.BlockSpec((3, Rn, C), lambda i: (0, i, 0)),
            pl.BlockSpec((C, C), lambda i: (0, 0)),
            pl.BlockSpec((C, C), lambda i: (0, 0)),
        ],
        out_specs=pl.BlockSpec((Rn, C), lambda i: (i, 0)),
        out_shape=jax.ShapeDtypeStruct((N, C), f32),
    )(pxn3, wi, wj)


def _sc_body(indi_hbm, indj_hbm, i1_hbm, d0_hbm, d1_hbm, d2_hbm,
             ti0, ti1, ti2, ti3, tj0, tj1, tj2, tj3,
             ix_hbm, pxnew_hbm,
             idxi_v, idxj_v, gi_v, gj_v, i1_v, diff_v, ixb_v, zb_v,
             acc_sh, sem_in0, sem_in1, sem_out0, sem_out1):
    cid = lax.axis_index("c")
    sid = lax.axis_index("s")
    r0 = sid * RPT
    sem_in = (sem_in0, sem_in1)
    sem_out = (sem_out0, sem_out1)
    d_hbm = (d0_hbm, d1_hbm, d2_hbm)

    def zrow(r, _):
        for x in range(3):
            for qq in range(CHQ // 16):
                zb_v[r, x, pl.ds(qq * 16, 16)] = jnp.zeros((16,), f32)
        return 0
    lax.fori_loop(0, ZR, zrow, 0)

    pltpu.sync_copy(indi_hbm.at[sid], idxi_v)
    pltpu.sync_copy(indj_hbm.at[sid], idxj_v)

    def make_edge_body(s):
        def edge_body(e):
            i1lo = i1_v[s, e, pl.ds(0, 16)]
            i1hi = i1_v[s, e, pl.ds(16, 16)]
            ev = jnp.full((16,), e, i32)
            for x in range(3):
                dv = plsc.load_gather(
                    diff_v,
                    [jnp.full((16,), s, i32), jnp.full((16,), x, i32), ev])
                gilo, gihi = plsc.unpack(
                    gi_v[s, e, x], format=plsc.PackFormat.INTERLEAVED)
                gjlo, gjhi = plsc.unpack(
                    gj_v[s, e, x], format=plsc.PackFormat.INTERLEAVED)
                ixb_v[s, e, x, pl.ds(0, 16)] = (gilo + gjlo + dv) * i1lo
                ixb_v[s, e, x, pl.ds(16, 16)] = (gihi + gjhi + dv) * i1hi
        return edge_body

    edge_bodies = (make_edge_body(0), make_edge_body(1))

    def in_copies(t, s, tab_i, tab_j, coff):
        base = sid * EPT + t * K
        return (
            (tab_i.at[idxi_v.at[t]], gi_v.at[s], sem_in[s]),
            (tab_j.at[idxj_v.at[t]], gj_v.at[s], sem_in[s]),
            (i1_hbm.at[sid, t, :, pl.ds(coff, CHQ)], i1_v.at[s], sem_in[s]),
            (d_hbm[0].at[pl.ds(base, K)], diff_v.at[s, 0], sem_in[s]),
            (d_hbm[1].at[pl.ds(base, K)], diff_v.at[s, 1], sem_in[s]),
            (d_hbm[2].at[pl.ds(base, K)], diff_v.at[s, 2], sem_in[s]),
        )

    def out_copies(t, s, coff):
        return tuple(
            (ixb_v.at[s, :, x, :],
             ix_hbm.at[x, sid, t, :, pl.ds(coff, CHQ)], sem_out[s])
            for x in range(3))


    def issue(cs):
        for src, dst, sem in cs:
            pltpu.async_copy(src, dst, sem)

    def drain(cs):
        for src, dst, sem in cs:
            pltpu.make_async_copy(src, dst, sem).wait()

    def run_pass(tab_i, tab_j, coff):
        issue(in_copies(0, 0, tab_i, tab_j, coff))

        def body2(tt, _):
            t0 = 2 * tt
            for s in range(2):
                t = t0 + s
                drain(in_copies(t, s, tab_i, tab_j, coff))
                nxt = t + 1

                @pl.when(nxt < NCHK)
                def _():
                    issue(in_copies(nxt, 1 - s, tab_i, tab_j, coff))

                @pl.when(tt > 0)
                def _():
                    drain(out_copies(t - 2, s, coff))

                plsc.parallel_loop(0, K, 1, unroll=4)(edge_bodies[s])
                issue(out_copies(t, s, coff))
                pltpu.sync_copy(ixb_v.at[s], acc_sh.at[idxi_v.at[t]],
                                add=True)
            return 0

        lax.fori_loop(0, NCHK // 2, body2, 0)
        drain(out_copies(NCHK - 2, 0, coff))
        drain(out_copies(NCHK - 1, 1, coff))

    for h in range(2):
        for z in range(RPT // ZR):
            pltpu.sync_copy(zb_v, acc_sh.at[pl.ds(r0 + z * ZR, ZR)])
        plsc.subcore_barrier()

        tabs = ((ti0, tj0, 0), (ti1, tj1, CHQ)) if h == 0 else \
               ((ti2, tj2, 2 * CHQ), (ti3, tj3, 3 * CHQ))

        @pl.when(cid == 0)
        def _():
            run_pass(tabs[0][0], tabs[0][1], tabs[0][2])

        @pl.when(cid == 1)
        def _():
            run_pass(tabs[1][0], tabs[1][1], tabs[1][2])

        plsc.subcore_barrier()

        for x in range(3):
            @pl.when(cid == 0)
            def _():
                pltpu.sync_copy(
                    acc_sh.at[pl.ds(r0, RPT), x, :],
                    pxnew_hbm.at[x, pl.ds(r0, RPT), pl.ds(tabs[0][2], CHQ)])

            @pl.when(cid == 1)
            def _():
                pltpu.sync_copy(
                    acc_sh.at[pl.ds(r0, RPT), x, :],
                    pxnew_hbm.at[x, pl.ds(r0, RPT), pl.ds(tabs[1][2], CHQ)])


def _run_sc(indi, indj, i1r, d0, d1, d2, tabs):
    mesh = plsc.VectorSubcoreMesh(core_axis_name="c", subcore_axis_name="s")
    k = pl.kernel(
        _sc_body,
        out_type=[
            jax.ShapeDtypeStruct((3, NS, NCHK, K, C), f32),
            jax.ShapeDtypeStruct((3, N, C), f32),
        ],
        mesh=mesh,
        compiler_params=pltpu.CompilerParams(use_tc_tiling_on_sc=False,
                                             needs_layout_passes=False),
        scratch_types=[
            pltpu.VMEM((NCHK, K), i32),
            pltpu.VMEM((NCHK, K), i32),
            pltpu.VMEM((2, K, 3, CHQ), jnp.bfloat16),
            pltpu.VMEM((2, K, 3, CHQ), jnp.bfloat16),
            pltpu.VMEM((2, K, CHQ), f32),
            pltpu.VMEM((2, 3, K), f32),
            pltpu.VMEM((2, K, 3, CHQ), f32),
            pltpu.VMEM((ZR, 3, CHQ), f32),
            pltpu.VMEM_SHARED((N, 3, CHQ), f32),
            pltpu.SemaphoreType.DMA,
            pltpu.SemaphoreType.DMA,
            pltpu.SemaphoreType.DMA,
            pltpu.SemaphoreType.DMA,
        ],
    )
    return k(indi, indj, i1r, d0, d1, d2, *tabs)


def kernel(ind_2, px, i1, diff, W_pi_i, W_pi_j, W_dot_i, W_dot_j):
    ind_i = ind_2[:, 0].reshape(NS, NCHK, K)
    ind_j = ind_2[:, 1].reshape(NS, NCHK, K)
    i1r = i1.reshape(NS, NCHK, K, C)
    d0, d1, d2 = diff[:, 0], diff[:, 1], diff[:, 2]

    pxf4 = jnp.pad(px.reshape(N * 3, C),
                   ((0, NPF - N * 3), (0, 0))).reshape(NPF // 4, 4 * C)
    wbds = _block_diag_quarters(W_pi_i, W_pi_j)
    tabs = _make_tabs(pxf4, wbds)
    tabs = [t.reshape(TROWS * C // (3 * CHQ), 3, CHQ) for t in tabs]

    ix3, pxn3 = _run_sc(ind_i, ind_j, i1r, d0, d1, d2, tabs)
    ix = jnp.transpose(ix3.reshape(3, E, C), (1, 0, 2))
    px_new = jnp.transpose(pxn3, (1, 0, 2))

    dotted = _make_dot(pxn3, W_dot_i, W_dot_j)
    return px_new, ix, dotted
